# baseline (speedup 1.0000x reference)
import jax
import jax.numpy as jnp
from jax import lax
from jax.experimental import pallas as pl
from jax.experimental.pallas import tpu as pltpu

FEA = 14
OUT = 7
SCALE = 1.0 / 16.0
NB = 32
PAD = 56


def _roi_kernel(coords_ref, t_ref, o_ref):
    nb = pl.program_id(0)

    r_row = lax.broadcasted_iota(jnp.int32, (PAD, FEA * FEA), 0)
    yx = lax.broadcasted_iota(jnp.int32, (PAD, FEA * FEA), 1)
    i = r_row // 8
    j = r_row - 8 * i
    y = yx // FEA
    x = yx - FEA * y
    ay = 7 * y
    ax = 7 * x

    r1 = lax.broadcasted_iota(jnp.int32, (PAD, 1), 0)
    i1 = r1 // 8
    j1 = r1 - 8 * i1
    i1f = i1.astype(jnp.float32)
    j1f = j1.astype(jnp.float32)
    valid1 = j1 < OUT
    C7E = jnp.float32((1.0 / 7.0) * (1.0 + 1e-6))

    def axis_mask(a7, qfull, a, L):
        qL = qfull * L
        return ((a7 - (7 * a - 6)) >= qL) & ((a7 - (7 * a + L)) < qL)

    def axis_den(qf, Lf):
        m1 = qf * Lf
        s = jnp.floor(m1 * C7E)
        e = jnp.floor((m1 + (Lf + 6.0)) * C7E)
        return jnp.maximum(e - s, 1.0)

    blocks = []
    for k in range(NB):
        base = 4 * (NB * nb + k)
        x1 = coords_ref[base + 0]
        y1 = coords_ref[base + 1]
        x2 = coords_ref[base + 2]
        y2 = coords_ref[base + 3]
        Lx = x2 - x1
        Ly = y2 - y1
        m = axis_mask(ay, i, y1, Ly) & axis_mask(ax, j, x1, Lx)
        dy = axis_den(i1f, Ly.astype(jnp.float32))
        dx = axis_den(j1f, Lx.astype(jnp.float32))
        recip = jnp.where(valid1, 1.0 / (dy * dx), 0.0)
        blocks.append(jnp.where(m, recip, 0.0))
    w_all = jnp.concatenate(blocks, axis=0)

    r = jnp.dot(w_all, t_ref[...], preferred_element_type=jnp.float32)
    o_ref[...] = r.reshape(NB, PAD, r.shape[-1])


def kernel(tensor, ROI):
    B, C, H, W = tensor.shape
    N = ROI.shape[0]
    t = tensor.reshape(B * C, H * W).T

    c = ROI[:, 1:] * SCALE
    x1 = jnp.clip(jnp.floor(c[:, 0]), 0, FEA)
    y1 = jnp.clip(jnp.floor(c[:, 1]), 0, FEA)
    x2 = jnp.clip(jnp.ceil(c[:, 2]), 0, FEA)
    y2 = jnp.clip(jnp.ceil(c[:, 3]), 0, FEA)
    coords = jnp.stack([x1, y1, x2, y2], axis=1).astype(jnp.int32).reshape(-1)

    out = pl.pallas_call(
        _roi_kernel,
        out_shape=jax.ShapeDtypeStruct((N, PAD, B * C), jnp.float32),
        grid_spec=pltpu.PrefetchScalarGridSpec(
            num_scalar_prefetch=1,
            grid=(N // NB,),
            in_specs=[
                pl.BlockSpec((H * W, B * C), lambda n, s: (0, 0)),
            ],
            out_specs=pl.BlockSpec((NB, PAD, B * C), lambda n, s: (n, 0, 0)),
        ),
        compiler_params=pltpu.CompilerParams(
            dimension_semantics=("parallel",),
            vmem_limit_bytes=56 * 1024 * 1024,
        ),
        name="roi_adaptive_pool",
    )(coords, t)

    return out.reshape(N * B, OUT, 8, C)[:, :, :OUT].transpose(0, 3, 1, 2)


# speedup vs baseline: 1.0020x; 1.0020x over previous
import jax
import jax.numpy as jnp
from jax import lax
from jax.experimental import pallas as pl
from jax.experimental.pallas import tpu as pltpu

FEA = 14
OUT = 7
SCALE = 1.0 / 16.0
NB = 16
PAD = 56


def _roi_kernel(coords_ref, t_ref, o_ref):
    nb = pl.program_id(0)

    r_row = lax.broadcasted_iota(jnp.int32, (PAD, FEA * FEA), 0)
    yx = lax.broadcasted_iota(jnp.int32, (PAD, FEA * FEA), 1)
    i = r_row // 8
    j = r_row - 8 * i
    y = yx // FEA
    x = yx - FEA * y
    ay = 7 * y
    ax = 7 * x

    r1 = lax.broadcasted_iota(jnp.int32, (PAD, 1), 0)
    i1 = r1 // 8
    j1 = r1 - 8 * i1
    i1f = i1.astype(jnp.float32)
    j1f = j1.astype(jnp.float32)
    valid1 = j1 < OUT
    C7E = jnp.float32((1.0 / 7.0) * (1.0 + 1e-6))

    def axis_mask(a7, qfull, a, L):
        qL = qfull * L
        return ((a7 - (7 * a - 6)) >= qL) & ((a7 - (7 * a + L)) < qL)

    def axis_den(qf, Lf):
        m1 = qf * Lf
        s = jnp.floor(m1 * C7E)
        e = jnp.floor((m1 + (Lf + 6.0)) * C7E)
        return jnp.maximum(e - s, 1.0)

    blocks = []
    for k in range(NB):
        base = 4 * (NB * nb + k)
        x1 = coords_ref[base + 0]
        y1 = coords_ref[base + 1]
        x2 = coords_ref[base + 2]
        y2 = coords_ref[base + 3]
        Lx = x2 - x1
        Ly = y2 - y1
        m = axis_mask(ay, i, y1, Ly) & axis_mask(ax, j, x1, Lx)
        dy = axis_den(i1f, Ly.astype(jnp.float32))
        dx = axis_den(j1f, Lx.astype(jnp.float32))
        recip = jnp.where(valid1, 1.0 / (dy * dx), 0.0)
        blocks.append(jnp.where(m, recip, 0.0))
    w_all = jnp.concatenate(blocks, axis=0)

    r = jnp.dot(w_all, t_ref[...], preferred_element_type=jnp.float32)
    o_ref[...] = r.reshape(NB, PAD, r.shape[-1])


def kernel(tensor, ROI):
    B, C, H, W = tensor.shape
    N = ROI.shape[0]
    t = tensor.reshape(B * C, H * W).T

    c = ROI[:, 1:] * SCALE
    x1 = jnp.clip(jnp.floor(c[:, 0]), 0, FEA)
    y1 = jnp.clip(jnp.floor(c[:, 1]), 0, FEA)
    x2 = jnp.clip(jnp.ceil(c[:, 2]), 0, FEA)
    y2 = jnp.clip(jnp.ceil(c[:, 3]), 0, FEA)
    coords = jnp.stack([x1, y1, x2, y2], axis=1).astype(jnp.int32).reshape(-1)

    out = pl.pallas_call(
        _roi_kernel,
        out_shape=jax.ShapeDtypeStruct((N, PAD, B * C), jnp.float32),
        grid_spec=pltpu.PrefetchScalarGridSpec(
            num_scalar_prefetch=1,
            grid=(N // NB,),
            in_specs=[
                pl.BlockSpec((H * W, B * C), lambda n, s: (0, 0)),
            ],
            out_specs=pl.BlockSpec((NB, PAD, B * C), lambda n, s: (n, 0, 0)),
        ),
        compiler_params=pltpu.CompilerParams(
            dimension_semantics=("parallel",),
        ),
        name="roi_adaptive_pool",
    )(coords, t)

    return out.reshape(N * B, OUT, 8, C)[:, :, :OUT].transpose(0, 3, 1, 2)
